# Initial kernel scaffold; baseline (speedup 1.0000x reference)
import jax
import jax.numpy as jnp
from jax.experimental import pallas as pl


def kernel(x, edge_index, edge_attr, W1, b1, W2, b2):
    raise NotImplementedError("write your pallas kernel here")


# speedup vs baseline: 20.9528x; 20.9528x over previous
import functools

import jax
import jax.numpy as jnp
from jax import lax
from jax.experimental import pallas as pl
from jax.experimental.pallas import tpu as pltpu
from jax.experimental.pallas import tpu_sc as plsc

N_NODES = 10000
N_EDGES = 320000
D_FEAT = 128
HIDDEN = 16
N_OUT = 16

NC = 2
NS = 16
NW = NC * NS
LANES = 16
CHUNK = 128
NCHUNKS = N_EDGES // CHUNK
CPW = (NCHUNKS + NW - 1) // NW
NPAD = 10240
RPT = NPAD // NS

_F32 = jnp.float32


def _mesh():
    return plsc.VectorSubcoreMesh(
        core_axis_name="c", subcore_axis_name="s",
        num_cores=NC, num_subcores=NS)


def _zero_shared(zeros_v, acc_sh, s):
    def fill(i, carry):
        zeros_v[i, :] = jnp.zeros((LANES,), _F32)
        return carry
    lax.fori_loop(0, RPT, fill, 0)
    pltpu.sync_copy(zeros_v, acc_sh.at[pl.ds(s * RPT, RPT)])


@functools.partial(
    pl.kernel,
    out_type=jax.ShapeDtypeStruct((NC, NPAD, HIDDEN), _F32),
    mesh=_mesh(),
    compiler_params=pltpu.CompilerParams(use_tc_tiling_on_sc=False),
    scratch_types=[
        pltpu.VMEM((1, CHUNK), jnp.int32),
        pltpu.VMEM((CHUNK, HIDDEN), _F32),
        pltpu.VMEM((RPT, HIDDEN), _F32),
        pltpu.VMEM_SHARED((NPAD, HIDDEN), _F32),
    ],
)
def _sc_degree(dst_hbm, out_hbm, didx, ones_v, zeros_v, acc_sh):
    c = lax.axis_index("c")
    s = lax.axis_index("s")
    w = c * NS + s

    def fill_ones(i, carry):
        ones_v[i, :] = jnp.full((LANES,), 1.0, _F32)
        return carry
    lax.fori_loop(0, CHUNK, fill_ones, 0)
    _zero_shared(zeros_v, acc_sh, s)
    plsc.subcore_barrier()

    def chunk_body(i, carry):
        g = w + NW * i

        @pl.when(g < NCHUNKS)
        def _():
            pltpu.sync_copy(dst_hbm.at[pl.ds(g * CHUNK, CHUNK)], didx.at[0])
            pltpu.sync_copy(ones_v, acc_sh.at[didx.at[0]], add=True)
        return carry
    lax.fori_loop(0, CPW, chunk_body, 0)
    plsc.subcore_barrier()
    pltpu.sync_copy(acc_sh.at[pl.ds(s * RPT, RPT)],
                    out_hbm.at[c, pl.ds(s * RPT, RPT)])


@functools.partial(
    pl.kernel,
    out_type=jax.ShapeDtypeStruct((NC, NPAD, HIDDEN), _F32),
    mesh=_mesh(),
    compiler_params=pltpu.CompilerParams(use_tc_tiling_on_sc=False),
    scratch_types=[
        pltpu.VMEM((CHUNK,), jnp.int32),
        pltpu.VMEM((1, CHUNK), jnp.int32),
        pltpu.VMEM((CHUNK, HIDDEN), _F32),
        pltpu.VMEM((RPT, HIDDEN), _F32),
        pltpu.VMEM_SHARED((NPAD, HIDDEN), _F32),
        pltpu.SemaphoreType.DMA,
    ],
)
def _sc_aggregate(src_hbm, dst_hbm, hn_hbm, out_hbm,
                  sidx, didx, rows, zeros_v, acc_sh, sem):
    c = lax.axis_index("c")
    s = lax.axis_index("s")
    w = c * NS + s

    _zero_shared(zeros_v, acc_sh, s)
    plsc.subcore_barrier()

    def chunk_body(i, carry):
        g = w + NW * i

        @pl.when(g < NCHUNKS)
        def _():
            pltpu.sync_copy(src_hbm.at[pl.ds(g * CHUNK, CHUNK)], sidx)
            pltpu.sync_copy(dst_hbm.at[pl.ds(g * CHUNK, CHUNK)], didx.at[0])
            pltpu.async_copy(hn_hbm.at[sidx], rows, sem).wait()
            pltpu.sync_copy(rows, acc_sh.at[didx.at[0]], add=True)
        return carry
    lax.fori_loop(0, CPW, chunk_body, 0)
    plsc.subcore_barrier()
    pltpu.sync_copy(acc_sh.at[pl.ds(s * RPT, RPT)],
                    out_hbm.at[c, pl.ds(s * RPT, RPT)])


def _tc_first(x_ref, w1_ref, dm_ref, hn_ref, dis_ref):
    h = jnp.dot(x_ref[...], w1_ref[...], preferred_element_type=_F32)
    deg = 1.0 + dm_ref[0] + dm_ref[1]
    dis = lax.rsqrt(deg)
    dis_ref[...] = dis
    hn_ref[...] = dis * h


def _tc_mid(ag_ref, hn_ref, dis_ref, b1_ref, w2_ref, hn2_ref):
    dis = dis_ref[...]
    out1 = dis * (ag_ref[0] + ag_ref[1] + hn_ref[...]) + b1_ref[...]
    h2 = jnp.dot(out1, w2_ref[...], preferred_element_type=_F32)
    hn2_ref[...] = dis * h2


def _tc_last(ag_ref, hn2_ref, dis_ref, b2_ref, o_ref):
    y = dis_ref[...] * (ag_ref[0] + ag_ref[1] + hn2_ref[...]) + b2_ref[...]
    m = jnp.max(y, axis=-1, keepdims=True)
    lse = jnp.log(jnp.sum(jnp.exp(y - m), axis=-1, keepdims=True)) + m
    o_ref[...] = y - lse


_nh = jax.ShapeDtypeStruct((N_NODES, HIDDEN), _F32)

_tc_first_call = pl.pallas_call(_tc_first, out_shape=(_nh, _nh))
_tc_mid_call = pl.pallas_call(_tc_mid, out_shape=_nh)
_tc_last_call = pl.pallas_call(
    _tc_last, out_shape=jax.ShapeDtypeStruct((N_NODES, N_OUT), _F32))


def kernel(x, edge_index, edge_attr, W1, b1, W2, b2):
    del edge_attr
    src = edge_index[0].astype(jnp.int32)
    dst = edge_index[1].astype(jnp.int32)
    dm = _sc_degree(dst)[:, :N_NODES]
    hn1, dis = _tc_first_call(x, W1, dm)
    ag1 = _sc_aggregate(src, dst, hn1)[:, :N_NODES]
    hn2 = _tc_mid_call(ag1, hn1, dis, b1.reshape(1, HIDDEN), W2)
    ag2 = _sc_aggregate(src, dst, hn2)[:, :N_NODES]
    return _tc_last_call(ag2, hn2, dis, b2.reshape(1, N_OUT))


# speedup vs baseline: 48.6124x; 2.3201x over previous
import functools

import jax
import jax.numpy as jnp
from jax import lax
from jax.experimental import pallas as pl
from jax.experimental.pallas import tpu as pltpu
from jax.experimental.pallas import tpu_sc as plsc

N_NODES = 10000
N_EDGES = 320000
D_FEAT = 128
HIDDEN = 16
N_OUT = 16

NC = 2
NS = 16
NW = NC * NS
LANES = 16
CHUNK = 128
NCHUNKS = N_EDGES // CHUNK
CPW = (NCHUNKS + NW - 1) // NW
NREM = NCHUNKS - (CPW - 1) * NW
E_PAD = NW * CPW * CHUNK
NPAD = 10240
RPT = NPAD // NS

_F32 = jnp.float32


def _mesh():
    return plsc.VectorSubcoreMesh(
        core_axis_name="c", subcore_axis_name="s",
        num_cores=NC, num_subcores=NS)


def _tile_range(w):
    n = jnp.where(w < NREM, CPW, CPW - 1)
    start = w * (CPW - 1) + jnp.minimum(w, NREM)
    return start, n


def _zero_shared(zeros_v, acc_sh, s):
    def fill(i, carry):
        zeros_v[i, :] = jnp.zeros((LANES,), _F32)
        return carry
    lax.fori_loop(0, RPT, fill, 0)
    pltpu.sync_copy(zeros_v, acc_sh.at[pl.ds(s * RPT, RPT)])


@functools.partial(
    pl.kernel,
    out_type=jax.ShapeDtypeStruct((NC, NPAD, HIDDEN), _F32),
    mesh=_mesh(),
    compiler_params=pltpu.CompilerParams(use_tc_tiling_on_sc=False),
    scratch_types=[
        pltpu.VMEM((CPW, CHUNK), jnp.int32),
        pltpu.VMEM((CHUNK, HIDDEN), _F32),
        pltpu.VMEM((RPT, HIDDEN), _F32),
        pltpu.VMEM_SHARED((NPAD, HIDDEN), _F32),
        pltpu.SemaphoreType.DMA,
    ],
)
def _sc_degree(dst_hbm, out_hbm, didx, ones_v, zeros_v, acc_sh, ssem):
    c = lax.axis_index("c")
    s = lax.axis_index("s")
    w = c * NS + s
    start, nch = _tile_range(w)

    def fill_ones(i, carry):
        ones_v[i, :] = jnp.full((LANES,), 1.0, _F32)
        return carry
    lax.fori_loop(0, CHUNK, fill_ones, 0)
    _zero_shared(zeros_v, acc_sh, s)
    pltpu.sync_copy(dst_hbm.at[pl.ds(start, CPW)], didx)
    plsc.subcore_barrier()

    def fire(j, carry):
        @pl.when(j < nch)
        def _():
            pltpu.async_copy(ones_v, acc_sh.at[didx.at[j]], ssem, add=True)
        return carry
    lax.fori_loop(0, CPW, fire, 0)

    def drain(j, carry):
        @pl.when(j < nch)
        def _():
            pltpu.make_async_copy(ones_v, acc_sh.at[didx.at[0]], ssem).wait()
        return carry
    lax.fori_loop(0, CPW, drain, 0)

    plsc.subcore_barrier()
    pltpu.sync_copy(acc_sh.at[pl.ds(s * RPT, RPT)],
                    out_hbm.at[c, pl.ds(s * RPT, RPT)])


@functools.partial(
    pl.kernel,
    out_type=jax.ShapeDtypeStruct((NC, NPAD, HIDDEN), _F32),
    mesh=_mesh(),
    compiler_params=pltpu.CompilerParams(use_tc_tiling_on_sc=False),
    scratch_types=[
        pltpu.VMEM((CPW, CHUNK), jnp.int32),
        pltpu.VMEM((CPW, CHUNK), jnp.int32),
        pltpu.VMEM((CHUNK, HIDDEN), _F32),
        pltpu.VMEM((CHUNK, HIDDEN), _F32),
        pltpu.VMEM((RPT, HIDDEN), _F32),
        pltpu.VMEM_SHARED((NPAD, HIDDEN), _F32),
        pltpu.SemaphoreType.DMA,
        pltpu.SemaphoreType.DMA,
    ],
)
def _sc_aggregate(src_hbm, dst_hbm, hn_hbm, out_hbm,
                  sidx, didx, rows_a, rows_b, zeros_v, acc_sh,
                  gsem_a, gsem_b):
    c = lax.axis_index("c")
    s = lax.axis_index("s")
    w = c * NS + s
    start, nch = _tile_range(w)

    _zero_shared(zeros_v, acc_sh, s)
    pltpu.sync_copy(src_hbm.at[pl.ds(start, CPW)], sidx)
    pltpu.sync_copy(dst_hbm.at[pl.ds(start, CPW)], didx)
    plsc.subcore_barrier()

    @pl.when(nch > 0)
    def _():
        pltpu.async_copy(hn_hbm.at[sidx.at[0]], rows_a, gsem_a)

    def pair_body(p, carry):
        j0 = 2 * p
        j1 = j0 + 1

        @pl.when(j1 < nch)
        def _():
            pltpu.async_copy(hn_hbm.at[sidx.at[j1]], rows_b, gsem_b)

        @pl.when(j0 < nch)
        def _():
            pltpu.make_async_copy(hn_hbm.at[sidx.at[0]], rows_a, gsem_a).wait()
            pltpu.sync_copy(rows_a, acc_sh.at[didx.at[j0]], add=True)

        @pl.when(j0 + 2 < nch)
        def _():
            pltpu.async_copy(hn_hbm.at[sidx.at[j0 + 2]], rows_a, gsem_a)

        @pl.when(j1 < nch)
        def _():
            pltpu.make_async_copy(hn_hbm.at[sidx.at[0]], rows_b, gsem_b).wait()
            pltpu.sync_copy(rows_b, acc_sh.at[didx.at[j1]], add=True)
        return carry
    lax.fori_loop(0, (CPW + 1) // 2, pair_body, 0)

    plsc.subcore_barrier()
    pltpu.sync_copy(acc_sh.at[pl.ds(s * RPT, RPT)],
                    out_hbm.at[c, pl.ds(s * RPT, RPT)])


def _tc_first(x_ref, w1_ref, dm_ref, hn_ref, dis_ref):
    h = jnp.dot(x_ref[...], w1_ref[...], preferred_element_type=_F32)
    deg = 1.0 + dm_ref[0, :N_NODES] + dm_ref[1, :N_NODES]
    dis = lax.rsqrt(deg)
    dis_ref[...] = dis
    hn_ref[...] = dis * h


def _tc_mid(ag_ref, hn_ref, dis_ref, b1_ref, w2_ref, hn2_ref):
    dis = dis_ref[...]
    out1 = dis * (ag_ref[0, :N_NODES] + ag_ref[1, :N_NODES]
                  + hn_ref[...]) + b1_ref[...]
    h2 = jnp.dot(out1, w2_ref[...], preferred_element_type=_F32)
    hn2_ref[...] = dis * h2


def _tc_last(ag_ref, hn2_ref, dis_ref, b2_ref, o_ref):
    y = dis_ref[...] * (ag_ref[0, :N_NODES] + ag_ref[1, :N_NODES]
                        + hn2_ref[...]) + b2_ref[...]
    m = jnp.max(y, axis=-1, keepdims=True)
    lse = jnp.log(jnp.sum(jnp.exp(y - m), axis=-1, keepdims=True)) + m
    o_ref[...] = y - lse


_nh = jax.ShapeDtypeStruct((N_NODES, HIDDEN), _F32)

_tc_first_call = pl.pallas_call(_tc_first, out_shape=(_nh, _nh))
_tc_mid_call = pl.pallas_call(_tc_mid, out_shape=_nh)
_tc_last_call = pl.pallas_call(
    _tc_last, out_shape=jax.ShapeDtypeStruct((N_NODES, N_OUT), _F32))


def kernel(x, edge_index, edge_attr, W1, b1, W2, b2):
    del edge_attr
    ei = edge_index.astype(jnp.int32)
    src = jnp.pad(ei[0], (0, E_PAD - N_EDGES)).reshape(NW * CPW, CHUNK)
    dst = jnp.pad(ei[1], (0, E_PAD - N_EDGES)).reshape(NW * CPW, CHUNK)
    dm = _sc_degree(dst)
    hn1, dis = _tc_first_call(x, W1, dm)
    ag1 = _sc_aggregate(src, dst, hn1)
    hn2 = _tc_mid_call(ag1, hn1, dis, b1.reshape(1, HIDDEN), W2)
    ag2 = _sc_aggregate(src, dst, hn2)
    return _tc_last_call(ag2, hn2, dis, b2.reshape(1, N_OUT))


# speedup vs baseline: 60.9779x; 1.2544x over previous
import functools

import jax
import jax.numpy as jnp
from jax import lax
from jax.experimental import pallas as pl
from jax.experimental.pallas import tpu as pltpu
from jax.experimental.pallas import tpu_sc as plsc

N_NODES = 10000
N_EDGES = 320000
D_FEAT = 128
HIDDEN = 16
N_OUT = 16

NC = 2
NS = 16
NW = NC * NS
LANES = 16
CHUNK = 128
NCHUNKS = N_EDGES // CHUNK
CPW = (NCHUNKS + NW - 1) // NW
NREM = NCHUNKS - (CPW - 1) * NW
E_PAD = NW * CPW * CHUNK
NPAD = 10240
RPT = NPAD // NS

_F32 = jnp.float32


def _mesh():
    return plsc.VectorSubcoreMesh(
        core_axis_name="c", subcore_axis_name="s",
        num_cores=NC, num_subcores=NS)


def _tile_range(w):
    n = jnp.where(w < NREM, CPW, CPW - 1)
    start = w * (CPW - 1) + jnp.minimum(w, NREM)
    return start, n


def _zero_shared(zeros_v, acc_sh, s):
    def fill(i, carry):
        zeros_v[i, :] = jnp.zeros((LANES,), _F32)
        return carry
    lax.fori_loop(0, RPT, fill, 0)
    pltpu.sync_copy(zeros_v, acc_sh.at[pl.ds(s * RPT, RPT)])


@functools.partial(
    pl.kernel,
    out_type=jax.ShapeDtypeStruct((NC, NPAD, HIDDEN), _F32),
    mesh=_mesh(),
    compiler_params=pltpu.CompilerParams(use_tc_tiling_on_sc=False),
    scratch_types=[
        pltpu.VMEM((CPW, CHUNK), jnp.int32),
        pltpu.VMEM((CHUNK, HIDDEN), _F32),
        pltpu.VMEM((RPT, HIDDEN), _F32),
        pltpu.VMEM_SHARED((NPAD, HIDDEN), _F32),
        pltpu.SemaphoreType.DMA,
    ],
)
def _sc_degree(dst_hbm, out_hbm, didx, ones_v, zeros_v, acc_sh, ssem):
    c = lax.axis_index("c")
    s = lax.axis_index("s")
    w = c * NS + s
    start, nch = _tile_range(w)

    def fill_ones(i, carry):
        ones_v[i, :] = jnp.full((LANES,), 1.0, _F32)
        return carry
    lax.fori_loop(0, CHUNK, fill_ones, 0)
    _zero_shared(zeros_v, acc_sh, s)
    pltpu.sync_copy(dst_hbm.at[pl.ds(start, CPW)], didx)
    plsc.subcore_barrier()

    def fire(j, carry):
        @pl.when(j < nch)
        def _():
            pltpu.async_copy(ones_v, acc_sh.at[didx.at[j]], ssem, add=True)
        return carry
    lax.fori_loop(0, CPW, fire, 0)

    def drain(j, carry):
        @pl.when(j < nch)
        def _():
            pltpu.make_async_copy(ones_v, acc_sh.at[didx.at[0]], ssem).wait()
        return carry
    lax.fori_loop(0, CPW, drain, 0)

    plsc.subcore_barrier()
    pltpu.sync_copy(acc_sh.at[pl.ds(s * RPT, RPT)],
                    out_hbm.at[c, pl.ds(s * RPT, RPT)])


NB = 8
NGRP = (CPW + NB - 1) // NB


@functools.partial(
    pl.kernel,
    out_type=jax.ShapeDtypeStruct((NC, NPAD, HIDDEN), _F32),
    mesh=_mesh(),
    compiler_params=pltpu.CompilerParams(use_tc_tiling_on_sc=False),
    scratch_types=[
        pltpu.VMEM((CPW, CHUNK), jnp.int32),
        pltpu.VMEM((CPW, CHUNK), jnp.int32),
        pltpu.VMEM((NB, CHUNK, HIDDEN), _F32),
        pltpu.VMEM((RPT, HIDDEN), _F32),
        pltpu.VMEM_SHARED((NPAD, HIDDEN), _F32),
        pltpu.SemaphoreType.DMA((NB,)),
        pltpu.SemaphoreType.DMA((NB,)),
    ],
)
def _sc_aggregate(src_hbm, dst_hbm, hn_hbm, out_hbm,
                  sidx, didx, rows, zeros_v, acc_sh, gsem, ssem):
    c = lax.axis_index("c")
    s = lax.axis_index("s")
    w = c * NS + s
    start, nch = _tile_range(w)

    _zero_shared(zeros_v, acc_sh, s)
    pltpu.sync_copy(src_hbm.at[pl.ds(start, CPW)], sidx)
    pltpu.sync_copy(dst_hbm.at[pl.ds(start, CPW)], didx)
    plsc.subcore_barrier()

    def group_body(g, carry):
        base = g * NB
        for b in range(NB):
            j = base + b

            @pl.when(jnp.logical_and(j < nch, g > 0))
            def _():
                pltpu.make_async_copy(
                    rows.at[b], acc_sh.at[didx.at[0]], ssem.at[b]).wait()

            @pl.when(j < nch)
            def _():
                pltpu.async_copy(hn_hbm.at[sidx.at[j]], rows.at[b],
                                 gsem.at[b])
        for b in range(NB):
            j = base + b

            @pl.when(j < nch)
            def _():
                pltpu.make_async_copy(
                    hn_hbm.at[sidx.at[0]], rows.at[b], gsem.at[b]).wait()
                pltpu.async_copy(rows.at[b], acc_sh.at[didx.at[j]],
                                 ssem.at[b], add=True)
        return carry
    lax.fori_loop(0, NGRP, group_body, 0)
    for b in range(NB):
        pltpu.make_async_copy(
            rows.at[b], acc_sh.at[didx.at[0]], ssem.at[b]).wait()

    plsc.subcore_barrier()
    pltpu.sync_copy(acc_sh.at[pl.ds(s * RPT, RPT)],
                    out_hbm.at[c, pl.ds(s * RPT, RPT)])


def _tc_first(x_ref, w1_ref, dm_ref, hn_ref, dis_ref):
    h = jnp.dot(x_ref[...], w1_ref[...], preferred_element_type=_F32)
    deg = 1.0 + dm_ref[0, :N_NODES] + dm_ref[1, :N_NODES]
    dis = lax.rsqrt(deg)
    dis_ref[...] = dis
    hn_ref[...] = dis * h


def _tc_mid(ag_ref, hn_ref, dis_ref, b1_ref, w2_ref, hn2_ref):
    dis = dis_ref[...]
    out1 = dis * (ag_ref[0, :N_NODES] + ag_ref[1, :N_NODES]
                  + hn_ref[...]) + b1_ref[...]
    h2 = jnp.dot(out1, w2_ref[...], preferred_element_type=_F32)
    hn2_ref[...] = dis * h2


def _tc_last(ag_ref, hn2_ref, dis_ref, b2_ref, o_ref):
    y = dis_ref[...] * (ag_ref[0, :N_NODES] + ag_ref[1, :N_NODES]
                        + hn2_ref[...]) + b2_ref[...]
    m = jnp.max(y, axis=-1, keepdims=True)
    lse = jnp.log(jnp.sum(jnp.exp(y - m), axis=-1, keepdims=True)) + m
    o_ref[...] = y - lse


_nh = jax.ShapeDtypeStruct((N_NODES, HIDDEN), _F32)

_tc_first_call = pl.pallas_call(_tc_first, out_shape=(_nh, _nh))
_tc_mid_call = pl.pallas_call(_tc_mid, out_shape=_nh)
_tc_last_call = pl.pallas_call(
    _tc_last, out_shape=jax.ShapeDtypeStruct((N_NODES, N_OUT), _F32))


def kernel(x, edge_index, edge_attr, W1, b1, W2, b2):
    del edge_attr
    ei = edge_index.astype(jnp.int32)
    src = jnp.pad(ei[0], (0, E_PAD - N_EDGES)).reshape(NW * CPW, CHUNK)
    dst = jnp.pad(ei[1], (0, E_PAD - N_EDGES)).reshape(NW * CPW, CHUNK)
    dm = _sc_degree(dst)
    hn1, dis = _tc_first_call(x, W1, dm)
    ag1 = _sc_aggregate(src, dst, hn1)
    hn2 = _tc_mid_call(ag1, hn1, dis, b1.reshape(1, HIDDEN), W2)
    ag2 = _sc_aggregate(src, dst, hn2)
    return _tc_last_call(ag2, hn2, dis, b2.reshape(1, N_OUT))


# speedup vs baseline: 62.7787x; 1.0295x over previous
import functools

import jax
import jax.numpy as jnp
from jax import lax
from jax.experimental import pallas as pl
from jax.experimental.pallas import tpu as pltpu
from jax.experimental.pallas import tpu_sc as plsc

N_NODES = 10000
N_EDGES = 320000
D_FEAT = 128
HIDDEN = 16
N_OUT = 16

NC = 2
NS = 16
NW = NC * NS
LANES = 16
CHUNK = 128
NCHUNKS = N_EDGES // CHUNK
CPW = (NCHUNKS + NW - 1) // NW
NREM = NCHUNKS - (CPW - 1) * NW
NPAD = 10240
RPT = NPAD // NS
NB = 8
NGRP = (CPW + NB - 1) // NB
BLK = 1000

_F32 = jnp.float32


def _mesh():
    return plsc.VectorSubcoreMesh(
        core_axis_name="c", subcore_axis_name="s",
        num_cores=NC, num_subcores=NS)


def _tile_window(w):
    n = jnp.where(w < NREM, CPW, CPW - 1)
    start = w * (CPW - 1) + jnp.minimum(w, NREM)
    wstart = jnp.minimum(start, NCHUNKS - CPW)
    return wstart, start - wstart, n


def _zero_shared(zeros_v, acc_sh, s):
    def fill(i, carry):
        zeros_v[i, :] = jnp.zeros((LANES,), _F32)
        return carry
    lax.fori_loop(0, RPT, fill, 0)
    pltpu.sync_copy(zeros_v, acc_sh.at[pl.ds(s * RPT, RPT)])


@functools.partial(
    pl.kernel,
    out_type=jax.ShapeDtypeStruct((NC, NPAD, HIDDEN), _F32),
    mesh=_mesh(),
    compiler_params=pltpu.CompilerParams(use_tc_tiling_on_sc=False),
    scratch_types=[
        pltpu.VMEM((CPW, CHUNK), jnp.int32),
        pltpu.VMEM((CHUNK, HIDDEN), _F32),
        pltpu.VMEM((RPT, HIDDEN), _F32),
        pltpu.VMEM_SHARED((NPAD, HIDDEN), _F32),
        pltpu.SemaphoreType.DMA,
    ],
)
def _sc_degree(ei_hbm, out_hbm, didx, ones_v, zeros_v, acc_sh, ssem):
    c = lax.axis_index("c")
    s = lax.axis_index("s")
    w = c * NS + s
    wstart, off, nch = _tile_window(w)

    def fill_ones(i, carry):
        ones_v[i, :] = jnp.full((LANES,), 1.0, _F32)
        return carry
    lax.fori_loop(0, CHUNK, fill_ones, 0)
    _zero_shared(zeros_v, acc_sh, s)
    pltpu.sync_copy(ei_hbm.at[1, pl.ds(wstart, CPW)], didx)
    plsc.subcore_barrier()

    def fire(j, carry):
        @pl.when(j < nch)
        def _():
            pltpu.async_copy(ones_v, acc_sh.at[didx.at[j + off]], ssem,
                             add=True)
        return carry
    lax.fori_loop(0, CPW, fire, 0)

    def drain(j, carry):
        @pl.when(j < nch)
        def _():
            pltpu.make_async_copy(ones_v, acc_sh.at[didx.at[0]], ssem).wait()
        return carry
    lax.fori_loop(0, CPW, drain, 0)

    plsc.subcore_barrier()
    pltpu.sync_copy(acc_sh.at[pl.ds(s * RPT, RPT)],
                    out_hbm.at[c, pl.ds(s * RPT, RPT)])


@functools.partial(
    pl.kernel,
    out_type=jax.ShapeDtypeStruct((NC, NPAD, HIDDEN), _F32),
    mesh=_mesh(),
    compiler_params=pltpu.CompilerParams(use_tc_tiling_on_sc=False),
    scratch_types=[
        pltpu.VMEM((CPW, CHUNK), jnp.int32),
        pltpu.VMEM((CPW, CHUNK), jnp.int32),
        pltpu.VMEM((NB, CHUNK, HIDDEN), _F32),
        pltpu.VMEM((RPT, HIDDEN), _F32),
        pltpu.VMEM_SHARED((NPAD, HIDDEN), _F32),
        pltpu.SemaphoreType.DMA((NB,)),
        pltpu.SemaphoreType.DMA((NB,)),
    ],
)
def _sc_aggregate(ei_hbm, hn_hbm, out_hbm,
                  sidx, didx, rows, zeros_v, acc_sh, gsem, ssem):
    c = lax.axis_index("c")
    s = lax.axis_index("s")
    w = c * NS + s
    wstart, off, nch = _tile_window(w)

    _zero_shared(zeros_v, acc_sh, s)
    pltpu.sync_copy(ei_hbm.at[0, pl.ds(wstart, CPW)], sidx)
    pltpu.sync_copy(ei_hbm.at[1, pl.ds(wstart, CPW)], didx)
    plsc.subcore_barrier()

    def group_body(g, carry):
        base = g * NB
        for b in range(NB):
            j = base + b

            @pl.when(jnp.logical_and(j < nch, g > 0))
            def _():
                pltpu.make_async_copy(
                    rows.at[b], acc_sh.at[didx.at[0]], ssem.at[b]).wait()

            @pl.when(j < nch)
            def _():
                pltpu.async_copy(hn_hbm.at[sidx.at[j + off]], rows.at[b],
                                 gsem.at[b])
        for b in range(NB):
            j = base + b

            @pl.when(j < nch)
            def _():
                pltpu.make_async_copy(
                    hn_hbm.at[sidx.at[0]], rows.at[b], gsem.at[b]).wait()
                pltpu.async_copy(rows.at[b], acc_sh.at[didx.at[j + off]],
                                 ssem.at[b], add=True)
        return carry
    lax.fori_loop(0, NGRP, group_body, 0)
    for b in range(NB):
        pltpu.make_async_copy(
            rows.at[b], acc_sh.at[didx.at[0]], ssem.at[b]).wait()

    plsc.subcore_barrier()
    pltpu.sync_copy(acc_sh.at[pl.ds(s * RPT, RPT)],
                    out_hbm.at[c, pl.ds(s * RPT, RPT)])


def _tc_mm1(x_ref, w1_ref, h_ref):
    h_ref[...] = jnp.dot(x_ref[...], w1_ref[...],
                         preferred_element_type=_F32)


def _tc_scale(h_ref, dm_ref, hn_ref, dis_ref):
    deg = 1.0 + dm_ref[0] + dm_ref[1]
    dis = lax.rsqrt(deg)
    dis_ref[...] = dis
    hn_ref[...] = dis * h_ref[...]


def _tc_mid(ag_ref, hn_ref, dis_ref, b1_ref, w2_ref, hn2_ref):
    dis = dis_ref[...]
    out1 = dis * (ag_ref[0] + ag_ref[1] + hn_ref[...]) + b1_ref[...]
    h2 = jnp.dot(out1, w2_ref[...], preferred_element_type=_F32)
    hn2_ref[...] = dis * h2


def _tc_last(ag_ref, hn2_ref, dis_ref, b2_ref, o_ref):
    y = dis_ref[...] * (ag_ref[0] + ag_ref[1] + hn2_ref[...]) + b2_ref[...]
    m = jnp.max(y, axis=-1, keepdims=True)
    lse = jnp.log(jnp.sum(jnp.exp(y - m), axis=-1, keepdims=True)) + m
    o_ref[...] = y - lse


_GRID = N_NODES // BLK
_row_blk = pl.BlockSpec((BLK, HIDDEN), lambda i: (i, 0))
_row_blk_d = pl.BlockSpec((BLK, D_FEAT), lambda i: (i, 0))
_dm_blk = pl.BlockSpec((2, BLK, HIDDEN), lambda i: (0, i, 0))
_w1_blk = pl.BlockSpec((D_FEAT, HIDDEN), lambda i: (0, 0))
_w2_blk = pl.BlockSpec((HIDDEN, N_OUT), lambda i: (0, 0))
_b_blk = pl.BlockSpec((1, HIDDEN), lambda i: (0, 0))
_nh = jax.ShapeDtypeStruct((N_NODES, HIDDEN), _F32)

_tc_mm1_call = pl.pallas_call(
    _tc_mm1, grid=(_GRID,),
    in_specs=[_row_blk_d, _w1_blk], out_specs=_row_blk, out_shape=_nh)
_tc_scale_call = pl.pallas_call(
    _tc_scale, grid=(_GRID,),
    in_specs=[_row_blk, _dm_blk], out_specs=(_row_blk, _row_blk),
    out_shape=(_nh, _nh))
_tc_mid_call = pl.pallas_call(
    _tc_mid, grid=(_GRID,),
    in_specs=[_dm_blk, _row_blk, _row_blk, _b_blk, _w2_blk],
    out_specs=_row_blk, out_shape=_nh)
_tc_last_call = pl.pallas_call(
    _tc_last, grid=(_GRID,),
    in_specs=[_dm_blk, _row_blk, _row_blk, _b_blk],
    out_specs=_row_blk,
    out_shape=jax.ShapeDtypeStruct((N_NODES, N_OUT), _F32))


def kernel(x, edge_index, edge_attr, W1, b1, W2, b2):
    del edge_attr
    ei = edge_index.astype(jnp.int32).reshape(2, NCHUNKS, CHUNK)
    h1 = _tc_mm1_call(x, W1)
    dm = _sc_degree(ei)
    hn1, dis = _tc_scale_call(h1, dm)
    ag1 = _sc_aggregate(ei, hn1)
    hn2 = _tc_mid_call(ag1, hn1, dis, b1.reshape(1, HIDDEN), W2)
    ag2 = _sc_aggregate(ei, hn2)
    return _tc_last_call(ag2, hn2, dis, b2.reshape(1, N_OUT))


# speedup vs baseline: 65.8320x; 1.0486x over previous
import functools

import jax
import jax.numpy as jnp
from jax import lax
from jax.experimental import pallas as pl
from jax.experimental.pallas import tpu as pltpu
from jax.experimental.pallas import tpu_sc as plsc

N_NODES = 10000
N_EDGES = 320000
D_FEAT = 128
HIDDEN = 16
N_OUT = 16

NC = 2
NS = 16
NW = NC * NS
LANES = 16
CHUNK = 128
NCHUNKS = N_EDGES // CHUNK
CPW = (NCHUNKS + NW - 1) // NW
NREM = NCHUNKS - (CPW - 1) * NW
NPAD = 10240
RPT = NPAD // NS
NB = 12
NGRP = (CPW + NB - 1) // NB
BLK = 2000

_F32 = jnp.float32


def _mesh():
    return plsc.VectorSubcoreMesh(
        core_axis_name="c", subcore_axis_name="s",
        num_cores=NC, num_subcores=NS)


def _tile_window(w):
    n = jnp.where(w < NREM, CPW, CPW - 1)
    start = w * (CPW - 1) + jnp.minimum(w, NREM)
    wstart = jnp.minimum(start, NCHUNKS - CPW)
    return wstart, start - wstart, n


def _zero_shared(zeros_v, acc_sh, s):
    def fill(i, carry):
        zeros_v[i, :] = jnp.zeros((LANES,), _F32)
        return carry
    lax.fori_loop(0, RPT, fill, 0)
    pltpu.sync_copy(zeros_v, acc_sh.at[pl.ds(s * RPT, RPT)])


@functools.partial(
    pl.kernel,
    out_type=jax.ShapeDtypeStruct((NC, NPAD, HIDDEN), _F32),
    mesh=_mesh(),
    compiler_params=pltpu.CompilerParams(use_tc_tiling_on_sc=False),
    scratch_types=[
        pltpu.VMEM((CPW, CHUNK), jnp.int32),
        pltpu.VMEM((CHUNK, HIDDEN), _F32),
        pltpu.VMEM((RPT, HIDDEN), _F32),
        pltpu.VMEM_SHARED((NPAD, HIDDEN), _F32),
        pltpu.SemaphoreType.DMA,
    ],
)
def _sc_degree(ei_hbm, out_hbm, didx, ones_v, zeros_v, acc_sh, ssem):
    c = lax.axis_index("c")
    s = lax.axis_index("s")
    w = c * NS + s
    wstart, off, nch = _tile_window(w)

    def fill_ones(i, carry):
        ones_v[i, :] = jnp.full((LANES,), 1.0, _F32)
        return carry
    lax.fori_loop(0, CHUNK, fill_ones, 0)
    _zero_shared(zeros_v, acc_sh, s)
    pltpu.sync_copy(ei_hbm.at[1, pl.ds(wstart, CPW)], didx)
    plsc.subcore_barrier()

    def fire(j, carry):
        @pl.when(j < nch)
        def _():
            pltpu.async_copy(ones_v, acc_sh.at[didx.at[j + off]], ssem,
                             add=True)
        return carry
    lax.fori_loop(0, CPW, fire, 0)

    def drain(j, carry):
        @pl.when(j < nch)
        def _():
            pltpu.make_async_copy(ones_v, acc_sh.at[didx.at[0]], ssem).wait()
        return carry
    lax.fori_loop(0, CPW, drain, 0)

    plsc.subcore_barrier()
    pltpu.sync_copy(acc_sh.at[pl.ds(s * RPT, RPT)],
                    out_hbm.at[c, pl.ds(s * RPT, RPT)])


@functools.partial(
    pl.kernel,
    out_type=jax.ShapeDtypeStruct((NC, NPAD, HIDDEN), _F32),
    mesh=_mesh(),
    compiler_params=pltpu.CompilerParams(use_tc_tiling_on_sc=False),
    scratch_types=[
        pltpu.VMEM((CPW, CHUNK), jnp.int32),
        pltpu.VMEM((CPW, CHUNK), jnp.int32),
        pltpu.VMEM((NB, CHUNK, HIDDEN), _F32),
        pltpu.VMEM((RPT, HIDDEN), _F32),
        pltpu.VMEM_SHARED((NPAD, HIDDEN), _F32),
        pltpu.SemaphoreType.DMA((NB,)),
        pltpu.SemaphoreType.DMA((NB,)),
    ],
)
def _sc_aggregate(ei_hbm, hn_hbm, out_hbm,
                  sidx, didx, rows, zeros_v, acc_sh, gsem, ssem):
    c = lax.axis_index("c")
    s = lax.axis_index("s")
    w = c * NS + s
    wstart, off, nch = _tile_window(w)

    _zero_shared(zeros_v, acc_sh, s)
    pltpu.sync_copy(ei_hbm.at[0, pl.ds(wstart, CPW)], sidx)
    pltpu.sync_copy(ei_hbm.at[1, pl.ds(wstart, CPW)], didx)
    plsc.subcore_barrier()

    def group_body(g, carry):
        base = g * NB
        for b in range(NB):
            j = base + b

            @pl.when(jnp.logical_and(j < nch, g > 0))
            def _():
                pltpu.make_async_copy(
                    rows.at[b], acc_sh.at[didx.at[0]], ssem.at[b]).wait()

            @pl.when(j < nch)
            def _():
                pltpu.async_copy(hn_hbm.at[sidx.at[j + off]], rows.at[b],
                                 gsem.at[b])
        for b in range(NB):
            j = base + b

            @pl.when(j < nch)
            def _():
                pltpu.make_async_copy(
                    hn_hbm.at[sidx.at[0]], rows.at[b], gsem.at[b]).wait()
                pltpu.async_copy(rows.at[b], acc_sh.at[didx.at[j + off]],
                                 ssem.at[b], add=True)
        return carry
    lax.fori_loop(0, NGRP, group_body, 0)
    for b in range(NB):
        pltpu.make_async_copy(
            rows.at[b], acc_sh.at[didx.at[0]], ssem.at[b]).wait()

    plsc.subcore_barrier()
    pltpu.sync_copy(acc_sh.at[pl.ds(s * RPT, RPT)],
                    out_hbm.at[c, pl.ds(s * RPT, RPT)])


def _tc_mm1(x_ref, w1_ref, h_ref):
    h_ref[...] = jnp.dot(x_ref[...], w1_ref[...],
                         preferred_element_type=_F32)


def _tc_scale(h_ref, dm_ref, hn_ref, dis_ref):
    deg = 1.0 + dm_ref[0] + dm_ref[1]
    dis = lax.rsqrt(deg)
    dis_ref[...] = dis
    hn_ref[...] = dis * h_ref[...]


def _tc_mid(ag_ref, hn_ref, dis_ref, b1_ref, w2_ref, hn2_ref):
    dis = dis_ref[...]
    out1 = dis * (ag_ref[0] + ag_ref[1] + hn_ref[...]) + b1_ref[...]
    h2 = jnp.dot(out1, w2_ref[...], preferred_element_type=_F32)
    hn2_ref[...] = dis * h2


def _tc_last(ag_ref, hn2_ref, dis_ref, b2_ref, o_ref):
    y = dis_ref[...] * (ag_ref[0] + ag_ref[1] + hn2_ref[...]) + b2_ref[...]
    m = jnp.max(y, axis=-1, keepdims=True)
    lse = jnp.log(jnp.sum(jnp.exp(y - m), axis=-1, keepdims=True)) + m
    o_ref[...] = y - lse


_GRID = N_NODES // BLK
_row_blk = pl.BlockSpec((BLK, HIDDEN), lambda i: (i, 0))
_row_blk_d = pl.BlockSpec((BLK, D_FEAT), lambda i: (i, 0))
_dm_blk = pl.BlockSpec((2, BLK, HIDDEN), lambda i: (0, i, 0))
_w1_blk = pl.BlockSpec((D_FEAT, HIDDEN), lambda i: (0, 0))
_w2_blk = pl.BlockSpec((HIDDEN, N_OUT), lambda i: (0, 0))
_b_blk = pl.BlockSpec((1, HIDDEN), lambda i: (0, 0))
_nh = jax.ShapeDtypeStruct((N_NODES, HIDDEN), _F32)

_tc_mm1_call = pl.pallas_call(
    _tc_mm1, grid=(_GRID,),
    in_specs=[_row_blk_d, _w1_blk], out_specs=_row_blk, out_shape=_nh)
_tc_scale_call = pl.pallas_call(
    _tc_scale, grid=(_GRID,),
    in_specs=[_row_blk, _dm_blk], out_specs=(_row_blk, _row_blk),
    out_shape=(_nh, _nh))
_tc_mid_call = pl.pallas_call(
    _tc_mid, grid=(_GRID,),
    in_specs=[_dm_blk, _row_blk, _row_blk, _b_blk, _w2_blk],
    out_specs=_row_blk, out_shape=_nh)
_tc_last_call = pl.pallas_call(
    _tc_last, grid=(_GRID,),
    in_specs=[_dm_blk, _row_blk, _row_blk, _b_blk],
    out_specs=_row_blk,
    out_shape=jax.ShapeDtypeStruct((N_NODES, N_OUT), _F32))


def kernel(x, edge_index, edge_attr, W1, b1, W2, b2):
    del edge_attr
    ei = edge_index.astype(jnp.int32).reshape(2, NCHUNKS, CHUNK)
    h1 = _tc_mm1_call(x, W1)
    dm = _sc_degree(ei)
    hn1, dis = _tc_scale_call(h1, dm)
    ag1 = _sc_aggregate(ei, hn1)
    hn2 = _tc_mid_call(ag1, hn1, dis, b1.reshape(1, HIDDEN), W2)
    ag2 = _sc_aggregate(ei, hn2)
    return _tc_last_call(ag2, hn2, dis, b2.reshape(1, N_OUT))


# speedup vs baseline: 66.0015x; 1.0026x over previous
import functools

import jax
import jax.numpy as jnp
from jax import lax
from jax.experimental import pallas as pl
from jax.experimental.pallas import tpu as pltpu
from jax.experimental.pallas import tpu_sc as plsc

N_NODES = 10000
N_EDGES = 320000
D_FEAT = 128
HIDDEN = 16
N_OUT = 16

NC = 2
NS = 16
NW = NC * NS
LANES = 16
CHUNK = 256
NCHUNKS = N_EDGES // CHUNK
CPW = (NCHUNKS + NW - 1) // NW
NREM = NCHUNKS - (CPW - 1) * NW
NPAD = 10240
RPT = NPAD // NS
NB = 12
NGRP = (CPW + NB - 1) // NB
BLK = 2000

_F32 = jnp.float32


def _mesh():
    return plsc.VectorSubcoreMesh(
        core_axis_name="c", subcore_axis_name="s",
        num_cores=NC, num_subcores=NS)


def _tile_window(w):
    n = jnp.where(w < NREM, CPW, CPW - 1)
    start = w * (CPW - 1) + jnp.minimum(w, NREM)
    wstart = jnp.minimum(start, NCHUNKS - CPW)
    return wstart, start - wstart, n


def _zero_shared(zeros_v, acc_sh, s):
    def fill(i, carry):
        zeros_v[i, :] = jnp.zeros((LANES,), _F32)
        return carry
    lax.fori_loop(0, RPT, fill, 0)
    pltpu.sync_copy(zeros_v, acc_sh.at[pl.ds(s * RPT, RPT)])


@functools.partial(
    pl.kernel,
    out_type=jax.ShapeDtypeStruct((NC, NPAD, HIDDEN), _F32),
    mesh=_mesh(),
    compiler_params=pltpu.CompilerParams(use_tc_tiling_on_sc=False),
    scratch_types=[
        pltpu.VMEM((CPW, CHUNK), jnp.int32),
        pltpu.VMEM((CHUNK, HIDDEN), _F32),
        pltpu.VMEM((RPT, HIDDEN), _F32),
        pltpu.VMEM_SHARED((NPAD, HIDDEN), _F32),
        pltpu.SemaphoreType.DMA,
    ],
)
def _sc_degree(ei_hbm, out_hbm, didx, ones_v, zeros_v, acc_sh, ssem):
    c = lax.axis_index("c")
    s = lax.axis_index("s")
    w = c * NS + s
    wstart, off, nch = _tile_window(w)

    def fill_ones(i, carry):
        ones_v[i, :] = jnp.full((LANES,), 1.0, _F32)
        return carry
    lax.fori_loop(0, CHUNK, fill_ones, 0)
    _zero_shared(zeros_v, acc_sh, s)
    pltpu.sync_copy(ei_hbm.at[1, pl.ds(wstart, CPW)], didx)
    plsc.subcore_barrier()

    def fire(j, carry):
        @pl.when(j < nch)
        def _():
            pltpu.async_copy(ones_v, acc_sh.at[didx.at[j + off]], ssem,
                             add=True)
        return carry
    lax.fori_loop(0, CPW, fire, 0)

    def drain(j, carry):
        @pl.when(j < nch)
        def _():
            pltpu.make_async_copy(ones_v, acc_sh.at[didx.at[0]], ssem).wait()
        return carry
    lax.fori_loop(0, CPW, drain, 0)

    plsc.subcore_barrier()
    pltpu.sync_copy(acc_sh.at[pl.ds(s * RPT, RPT)],
                    out_hbm.at[c, pl.ds(s * RPT, RPT)])


@functools.partial(
    pl.kernel,
    out_type=jax.ShapeDtypeStruct((NC, NPAD, HIDDEN), _F32),
    mesh=_mesh(),
    compiler_params=pltpu.CompilerParams(use_tc_tiling_on_sc=False),
    scratch_types=[
        pltpu.VMEM((CPW, CHUNK), jnp.int32),
        pltpu.VMEM((CPW, CHUNK), jnp.int32),
        pltpu.VMEM((NB, CHUNK, HIDDEN), _F32),
        pltpu.VMEM((RPT, HIDDEN), _F32),
        pltpu.VMEM_SHARED((NPAD, HIDDEN), _F32),
        pltpu.SemaphoreType.DMA((NB,)),
        pltpu.SemaphoreType.DMA((NB,)),
    ],
)
def _sc_aggregate(ei_hbm, hn_hbm, out_hbm,
                  sidx, didx, rows, zeros_v, acc_sh, gsem, ssem):
    c = lax.axis_index("c")
    s = lax.axis_index("s")
    w = c * NS + s
    wstart, off, nch = _tile_window(w)

    _zero_shared(zeros_v, acc_sh, s)
    pltpu.sync_copy(ei_hbm.at[0, pl.ds(wstart, CPW)], sidx)
    pltpu.sync_copy(ei_hbm.at[1, pl.ds(wstart, CPW)], didx)
    plsc.subcore_barrier()

    def group_body(g, carry):
        base = g * NB
        for b in range(NB):
            j = base + b

            @pl.when(jnp.logical_and(j < nch, g > 0))
            def _():
                pltpu.make_async_copy(
                    rows.at[b], acc_sh.at[didx.at[0]], ssem.at[b]).wait()

            @pl.when(j < nch)
            def _():
                pltpu.async_copy(hn_hbm.at[sidx.at[j + off]], rows.at[b],
                                 gsem.at[b])
        for b in range(NB):
            j = base + b

            @pl.when(j < nch)
            def _():
                pltpu.make_async_copy(
                    hn_hbm.at[sidx.at[0]], rows.at[b], gsem.at[b]).wait()
                pltpu.async_copy(rows.at[b], acc_sh.at[didx.at[j + off]],
                                 ssem.at[b], add=True)
        return carry
    lax.fori_loop(0, NGRP, group_body, 0)
    for b in range(NB):
        pltpu.make_async_copy(
            rows.at[b], acc_sh.at[didx.at[0]], ssem.at[b]).wait()

    plsc.subcore_barrier()
    pltpu.sync_copy(acc_sh.at[pl.ds(s * RPT, RPT)],
                    out_hbm.at[c, pl.ds(s * RPT, RPT)])


def _tc_mm1(x_ref, w1_ref, h_ref):
    h_ref[...] = jnp.dot(x_ref[...], w1_ref[...],
                         preferred_element_type=_F32)


def _tc_scale(h_ref, dm_ref, hn_ref, dis_ref):
    deg = 1.0 + dm_ref[0] + dm_ref[1]
    dis = lax.rsqrt(deg)
    dis_ref[...] = dis
    hn_ref[...] = dis * h_ref[...]


def _tc_mid(ag_ref, hn_ref, dis_ref, b1_ref, w2_ref, hn2_ref):
    dis = dis_ref[...]
    out1 = dis * (ag_ref[0] + ag_ref[1] + hn_ref[...]) + b1_ref[...]
    h2 = jnp.dot(out1, w2_ref[...], preferred_element_type=_F32)
    hn2_ref[...] = dis * h2


def _tc_last(ag_ref, hn2_ref, dis_ref, b2_ref, o_ref):
    y = dis_ref[...] * (ag_ref[0] + ag_ref[1] + hn2_ref[...]) + b2_ref[...]
    m = jnp.max(y, axis=-1, keepdims=True)
    lse = jnp.log(jnp.sum(jnp.exp(y - m), axis=-1, keepdims=True)) + m
    o_ref[...] = y - lse


_GRID = N_NODES // BLK
_row_blk = pl.BlockSpec((BLK, HIDDEN), lambda i: (i, 0))
_row_blk_d = pl.BlockSpec((BLK, D_FEAT), lambda i: (i, 0))
_dm_blk = pl.BlockSpec((2, BLK, HIDDEN), lambda i: (0, i, 0))
_w1_blk = pl.BlockSpec((D_FEAT, HIDDEN), lambda i: (0, 0))
_w2_blk = pl.BlockSpec((HIDDEN, N_OUT), lambda i: (0, 0))
_b_blk = pl.BlockSpec((1, HIDDEN), lambda i: (0, 0))
_nh = jax.ShapeDtypeStruct((N_NODES, HIDDEN), _F32)

_tc_mm1_call = pl.pallas_call(
    _tc_mm1, grid=(_GRID,),
    in_specs=[_row_blk_d, _w1_blk], out_specs=_row_blk, out_shape=_nh)
_tc_scale_call = pl.pallas_call(
    _tc_scale, grid=(_GRID,),
    in_specs=[_row_blk, _dm_blk], out_specs=(_row_blk, _row_blk),
    out_shape=(_nh, _nh))
_tc_mid_call = pl.pallas_call(
    _tc_mid, grid=(_GRID,),
    in_specs=[_dm_blk, _row_blk, _row_blk, _b_blk, _w2_blk],
    out_specs=_row_blk, out_shape=_nh)
_tc_last_call = pl.pallas_call(
    _tc_last, grid=(_GRID,),
    in_specs=[_dm_blk, _row_blk, _row_blk, _b_blk],
    out_specs=_row_blk,
    out_shape=jax.ShapeDtypeStruct((N_NODES, N_OUT), _F32))


def kernel(x, edge_index, edge_attr, W1, b1, W2, b2):
    del edge_attr
    ei = edge_index.astype(jnp.int32).reshape(2, NCHUNKS, CHUNK)
    h1 = _tc_mm1_call(x, W1)
    dm = _sc_degree(ei)
    hn1, dis = _tc_scale_call(h1, dm)
    ag1 = _sc_aggregate(ei, hn1)
    hn2 = _tc_mid_call(ag1, hn1, dis, b1.reshape(1, HIDDEN), W2)
    ag2 = _sc_aggregate(ei, hn2)
    return _tc_last_call(ag2, hn2, dis, b2.reshape(1, N_OUT))
